# baseline (speedup 1.0000x reference)
import functools

import jax
import jax.numpy as jnp
import numpy as np
from jax import lax
from jax.experimental import pallas as pl
from jax.experimental.pallas import tpu as pltpu
from jax.experimental.pallas import tpu_sc as plsc

H = 8
D = 16
HID = 128
N = 10000
E = 320000
WID = HID + 16

NW = 32
CHUNK = 128
E_PAD = 327680
ROWS_PER_W = E_PAD // NW // CHUNK
NPAD = 10112
RPT = NPAD // 16
_CH = ((0, 128), (128, 128), (256, 128), (384, 128), (512, 120))

_f32 = jnp.float32
_i32 = jnp.int32


def _gelu(x):
    return 0.5 * x * (1.0 + lax.erf(x / np.sqrt(2.0).astype(np.float32)))


def _proj0_body(x0p, x0a, lwp, lbp, lwa, lba,
                wqp, bqp, wka, bka, wva, bva,
                wqa, bqa, wkp, bkp, wvp, bvp,
                xp, xa, qp, ka, va, qa, kp, vp):
    xpv = jnp.maximum(jnp.dot(x0p[...], lwp[...], preferred_element_type=_f32)
                      + lbp[...], 0.0)
    xav = jnp.maximum(jnp.dot(x0a[...], lwa[...], preferred_element_type=_f32)
                      + lba[...], 0.0)
    xp[...] = xpv
    xa[...] = xav
    qp[...] = jnp.dot(xpv, wqp[...], preferred_element_type=_f32) + bqp[...]
    ka[...] = jnp.dot(xav, wka[...], preferred_element_type=_f32) + bka[...]
    va[...] = jnp.dot(xav, wva[...], preferred_element_type=_f32) + bva[...]
    qa[...] = jnp.dot(xav, wqa[...], preferred_element_type=_f32) + bqa[...]
    kp[...] = jnp.dot(xpv, wkp[...], preferred_element_type=_f32) + bkp[...]
    vp[...] = jnp.dot(xpv, wvp[...], preferred_element_type=_f32) + bvp[...]


def _proj1_body(x0p, x0a,
                wqp, bqp, wka, bka, wva, bva,
                wqa, bqa, wkp, bkp, wvp, bvp,
                qp, ka, va, qa, kp, vp):
    xpv = x0p[...]
    xav = x0a[...]
    qp[...] = jnp.dot(xpv, wqp[...], preferred_element_type=_f32) + bqp[...]
    ka[...] = jnp.dot(xav, wka[...], preferred_element_type=_f32) + bka[...]
    va[...] = jnp.dot(xav, wva[...], preferred_element_type=_f32) + bva[...]
    qa[...] = jnp.dot(xav, wqa[...], preferred_element_type=_f32) + bqa[...]
    kp[...] = jnp.dot(xpv, wkp[...], preferred_element_type=_f32) + bkp[...]
    vp[...] = jnp.dot(xpv, wvp[...], preferred_element_type=_f32) + bvp[...]


_BM = 2000


def _run_proj0(x0p, x0a, ws):
    grid = (N // _BM,)
    xspec = pl.BlockSpec((_BM, HID), lambda i: (i, 0))
    wspec = pl.BlockSpec((HID, HID), lambda i: (0, 0))
    bspec = pl.BlockSpec((1, HID), lambda i: (0, 0))
    in_specs = [xspec, xspec] + [wspec if w.shape == (HID, HID) else bspec
                                 for w in ws]
    return pl.pallas_call(
        _proj0_body, grid=grid, in_specs=in_specs, out_specs=[xspec] * 8,
        out_shape=[jax.ShapeDtypeStruct((N, HID), _f32)] * 8)(x0p, x0a, *ws)


def _run_proj1(xp, xa, ws):
    grid = (N // _BM,)
    xspec = pl.BlockSpec((_BM, HID), lambda i: (i, 0))
    wspec = pl.BlockSpec((HID, HID), lambda i: (0, 0))
    bspec = pl.BlockSpec((1, HID), lambda i: (0, 0))
    in_specs = [xspec, xspec] + [wspec if w.shape == (HID, HID) else bspec
                                 for w in ws]
    return pl.pallas_call(
        _proj1_body, grid=grid, in_specs=in_specs, out_specs=[xspec] * 6,
        out_shape=[jax.ShapeDtypeStruct((N, HID), _f32)] * 6)(xp, xa, *ws)


_BE = 2048


def _edge_body(ke, qe, ve, e16, r16, msg, z128):
    p = ke[...] * qe[...]
    z = jnp.exp(jnp.dot(p, e16[...], preferred_element_type=_f32))
    zb = jnp.dot(z, r16[...], preferred_element_type=_f32)
    z128[...] = zb
    msg[...] = ve[...] * zb


def _run_edge(ke, qe, ve, e16, r16):
    grid = (E_PAD // _BE,)
    espec = pl.BlockSpec((_BE, HID), lambda i: (i, 0))
    return pl.pallas_call(
        _edge_body, grid=grid,
        in_specs=[espec, espec, espec,
                  pl.BlockSpec((HID, 16), lambda i: (0, 0)),
                  pl.BlockSpec((16, HID), lambda i: (0, 0))],
        out_specs=[espec, espec],
        out_shape=[jax.ShapeDtypeStruct((E_PAD, HID), _f32)] * 2)(ke, qe, ve,
                                                                  e16, r16)


def _epi_body(accm2, accz2, xin, aw, ab, skc, gbn, bbn, out):
    a = accm2[0:N, :] + accm2[NPAD:NPAD + N, :]
    szb = accz2[0:N, :] + accz2[NPAD:NPAD + N, :]
    aggn = a / (szb + 1e-16)
    g = _gelu(aggn)
    o = (jnp.dot(g, aw[...], preferred_element_type=_f32) + ab[...]
         + skc[...] * xin[...])
    h = _gelu(o) + xin[...]
    mu = jnp.mean(h, axis=0, keepdims=True)
    var = jnp.mean((h - mu) * (h - mu), axis=0, keepdims=True)
    out[...] = (h - mu) / jnp.sqrt(var + 1e-5) * gbn[...] + bbn[...]


def _run_epi(accm2, accz2, xin, aw, ab, skc, gbn, bbn):
    return pl.pallas_call(
        _epi_body,
        out_shape=jax.ShapeDtypeStruct((N, HID), _f32),
    )(accm2, accz2, xin, aw, ab, skc, gbn, bbn)


@functools.lru_cache(maxsize=None)
def _get_sc_gather():
    mesh = plsc.VectorSubcoreMesh(core_axis_name="c", subcore_axis_name="s")
    return functools.partial(
        pl.kernel,
        out_type=[jax.ShapeDtypeStruct((E_PAD, HID), _f32)] * 3,
        mesh=mesh,
        scratch_types=[
            pltpu.VMEM((1, CHUNK), _i32),
            pltpu.VMEM((1, CHUNK), _i32),
            pltpu.VMEM((1, CHUNK), _i32),
            pltpu.VMEM((1, CHUNK), _i32),
            pltpu.VMEM((CHUNK, HID), _f32),
            pltpu.VMEM((CHUNK, HID), _f32),
            pltpu.VMEM((CHUNK, HID), _f32),
            pltpu.SemaphoreType.DMA,
            pltpu.SemaphoreType.DMA,
            pltpu.SemaphoreType.DMA,
            pltpu.SemaphoreType.DMA,
        ],
    )(_sc_gather_body)


def _sc_gather(ktab, vtab, qtab, src2d, dst2d):
    return _get_sc_gather()(ktab, vtab, qtab, src2d, dst2d)


def _sc_gather_body(ktab, vtab, qtab, src2d, dst2d, ke, ve, qe,
                    sidx0, didx0, sidx1, didx1, kr, vr, qr,
                    semI0, semI1, semg, semw):
    c = lax.axis_index("c")
    s = lax.axis_index("s")
    wid = c * 16 + s
    row_base = wid * ROWS_PER_W

    pltpu.async_copy(src2d.at[pl.ds(row_base, 1)], sidx0, semI0)
    pltpu.async_copy(dst2d.at[pl.ds(row_base, 1)], didx0, semI0)

    bufs = ((sidx0, didx0, semI0), (sidx1, didx1, semI1))

    def body(j, carry):
        for b in (0, 1):
            i = j * 2 + b
            sidx, didx, semI = bufs[b]
            nsidx, ndidx, nsemI = bufs[1 - b]
            row = row_base + i
            base = pl.multiple_of(row * CHUNK, CHUNK)
            pltpu.make_async_copy(src2d.at[pl.ds(row, 1)], sidx, semI).wait()
            pltpu.make_async_copy(dst2d.at[pl.ds(row, 1)], didx, semI).wait()

            @pl.when(i + 1 < ROWS_PER_W)
            def _():
                pltpu.async_copy(src2d.at[pl.ds(row + 1, 1)], nsidx, nsemI)
                pltpu.async_copy(dst2d.at[pl.ds(row + 1, 1)], ndidx, nsemI)

            @pl.when(i > 0)
            def _():
                pltpu.make_async_copy(kr, ke.at[pl.ds(base, CHUNK)],
                                      semw).wait()
                pltpu.make_async_copy(vr, ve.at[pl.ds(base, CHUNK)],
                                      semw).wait()
                pltpu.make_async_copy(qr, qe.at[pl.ds(base, CHUNK)],
                                      semw).wait()

            pltpu.async_copy(ktab.at[sidx.at[0]], kr, semg)
            pltpu.async_copy(vtab.at[sidx.at[0]], vr, semg)
            pltpu.async_copy(qtab.at[didx.at[0]], qr, semg)
            pltpu.make_async_copy(ktab.at[sidx.at[0]], kr, semg).wait()
            pltpu.make_async_copy(vtab.at[sidx.at[0]], vr, semg).wait()
            pltpu.make_async_copy(qtab.at[didx.at[0]], qr, semg).wait()
            pltpu.async_copy(kr, ke.at[pl.ds(base, CHUNK)], semw)
            pltpu.async_copy(vr, ve.at[pl.ds(base, CHUNK)], semw)
            pltpu.async_copy(qr, qe.at[pl.ds(base, CHUNK)], semw)
        return carry

    lax.fori_loop(0, ROWS_PER_W // 2, body, 0)
    last = pl.multiple_of((row_base + ROWS_PER_W - 1) * CHUNK, CHUNK)
    pltpu.make_async_copy(kr, ke.at[pl.ds(last, CHUNK)], semw).wait()
    pltpu.make_async_copy(vr, ve.at[pl.ds(last, CHUNK)], semw).wait()
    pltpu.make_async_copy(qr, qe.at[pl.ds(last, CHUNK)], semw).wait()


@functools.lru_cache(maxsize=None)
def _get_sc_scatter():
    mesh = plsc.VectorSubcoreMesh(core_axis_name="c", subcore_axis_name="s")
    return functools.partial(
        pl.kernel,
        out_type=jax.ShapeDtypeStruct((2 * NPAD, HID), _f32),
        mesh=mesh,
        scratch_types=[
            pltpu.VMEM_SHARED((NPAD, HID), _f32),
            pltpu.VMEM((CHUNK, HID), _f32),
            pltpu.VMEM((CHUNK, HID), _f32),
            pltpu.VMEM((CHUNK,), _i32),
            pltpu.VMEM((CHUNK,), _i32),
            pltpu.SemaphoreType.DMA,
            pltpu.SemaphoreType.DMA,
        ],
    )(_sc_scatter_body)


def _sc_scatter(mz, dst2d, zeros_acc):
    return _get_sc_scatter()(mz, dst2d.reshape(-1), zeros_acc)


def _sc_scatter_body(mz, dst1d, zeros_acc, accout, acc, mv0, mv1, di0, di1,
                     semL0, semL1):
    c = lax.axis_index("c")
    s = lax.axis_index("s")
    wid = c * 16 + s
    r0 = s * RPT
    pltpu.sync_copy(zeros_acc.at[pl.ds(0, CHUNK)], mv0)
    for (o, n) in _CH:
        pltpu.sync_copy(mv0.at[pl.ds(0, n)], acc.at[pl.ds(r0 + o, n)])
    plsc.subcore_barrier()

    base0 = pl.multiple_of(wid * ROWS_PER_W * CHUNK, CHUNK)
    pltpu.async_copy(dst1d.at[pl.ds(base0, CHUNK)], di0, semL0)
    pltpu.async_copy(mz.at[pl.ds(base0, CHUNK)], mv0, semL0)

    bufs = ((mv0, di0, semL0), (mv1, di1, semL1))

    def body(j, carry):
        for b in (0, 1):
            i = j * 2 + b
            mv, di, semL = bufs[b]
            nmv, ndi, nsemL = bufs[1 - b]
            nbase = pl.multiple_of((wid * ROWS_PER_W + i + 1) * CHUNK, CHUNK)

            @pl.when(i + 1 < ROWS_PER_W)
            def _():
                pltpu.async_copy(dst1d.at[pl.ds(nbase, CHUNK)], ndi, nsemL)
                pltpu.async_copy(mz.at[pl.ds(nbase, CHUNK)], nmv, nsemL)

            pltpu.make_async_copy(dst1d.at[pl.ds(base0, CHUNK)], di,
                                  semL).wait()
            pltpu.make_async_copy(mz.at[pl.ds(base0, CHUNK)], mv, semL).wait()
            pltpu.sync_copy(mv, acc.at[di], add=True)
        return carry

    lax.fori_loop(0, ROWS_PER_W // 2, body, 0)
    plsc.subcore_barrier()
    out0 = c * NPAD + r0
    for (o, n) in _CH:
        pltpu.sync_copy(acc.at[pl.ds(r0 + o, n)], mv0.at[pl.ds(0, n)])
        pltpu.sync_copy(mv0.at[pl.ds(0, n)], accout.at[pl.ds(out0 + o, n)])


def _fold_rel(w, b, rel):
    w3 = w.reshape(HID, H, D)
    wf = jnp.einsum("chd,hde->che", w3, rel).reshape(HID, HID)
    bf = jnp.einsum("hd,hde->he", b.reshape(H, D), rel).reshape(HID)
    return wf, bf


def _pad_idx(idx, fill):
    idx = idx.astype(_i32)
    pad = jnp.full((E_PAD - E,), fill, _i32)
    return jnp.concatenate([idx, pad]).reshape(E_PAD // CHUNK, CHUNK)


def kernel(x_paper, x_author, edge_index_writes, edge_index_rev, params):
    p = params

    e16 = np.zeros((HID, 16), np.float32)
    r16 = np.zeros((16, HID), np.float32)
    for h in range(H):
        for d in range(D):
            e16[h * D + d, h] = 1.0
            r16[h, h * D + d] = 1.0
    e16 = jnp.asarray(e16)
    r16 = jnp.asarray(r16)

    zeros_acc = jnp.zeros((NPAD, HID), _f32)

    src_w = _pad_idx(edge_index_writes[0], 0)
    dst_w = _pad_idx(edge_index_writes[1], N)
    src_r = _pad_idx(edge_index_rev[0], 0)
    dst_r = _pad_idx(edge_index_rev[1], N)

    def b2(b):
        return b.reshape(1, HID)

    folded = {}
    for l in range(2):
        for (st, r, dt) in [("author", "writes", "paper"),
                            ("paper", "rev", "author")]:
            arel = p[f"arel_{r}_{l}"] * (p[f"prel_{r}_{l}"]
                                         / np.float32(np.sqrt(D)))[:, None, None]
            mrel = p[f"mrel_{r}_{l}"]
            folded[f"k_{r}_{l}"] = _fold_rel(p[f"k_w_{st}_{l}"],
                                             p[f"k_b_{st}_{l}"], arel)
            folded[f"v_{r}_{l}"] = _fold_rel(p[f"v_w_{st}_{l}"],
                                             p[f"v_b_{st}_{l}"], mrel)

    def proj_ws(l):
        kw_w, kb_w = folded[f"k_writes_{l}"]
        vw_w, vb_w = folded[f"v_writes_{l}"]
        kw_r, kb_r = folded[f"k_rev_{l}"]
        vw_r, vb_r = folded[f"v_rev_{l}"]
        return [p[f"q_w_paper_{l}"], b2(p[f"q_b_paper_{l}"]),
                kw_w, b2(kb_w), vw_w, b2(vb_w),
                p[f"q_w_author_{l}"], b2(p[f"q_b_author_{l}"]),
                kw_r, b2(kb_r), vw_r, b2(vb_r)]

    def epi_ws(l, t):
        sk = jax.nn.sigmoid(p[f"skip_{t}_{l}"])
        aw = sk * p[f"a_w_{t}_{l}"]
        ab = b2(sk * p[f"a_b_{t}_{l}"])
        skc = (1.0 - sk).reshape(1, 1)
        return aw, ab, skc, b2(p[f"bn_g_{t}_{l}"]), b2(p[f"bn_b_{t}_{l}"])

    xp, xa, qp, ka, va, qa, kp, vp = _run_proj0(
        x_paper, x_author,
        [p["lin_w_paper"], b2(p["lin_b_paper"]),
         p["lin_w_author"], b2(p["lin_b_author"])] + proj_ws(0))

    for l in range(2):
        if l == 1:
            qp, ka, va, qa, kp, vp = _run_proj1(xp, xa, proj_ws(1))

        ke_w, ve_w, qe_w = _sc_gather(ka, va, qp, src_w, dst_w)
        msg_w, zb_w = _run_edge(ke_w, qe_w, ve_w, e16, r16)
        accm_p = _sc_scatter(msg_w, dst_w, zeros_acc)
        accz_p = _sc_scatter(zb_w, dst_w, zeros_acc)

        ke_r, ve_r, qe_r = _sc_gather(kp, vp, qa, src_r, dst_r)
        msg_r, zb_r = _run_edge(ke_r, qe_r, ve_r, e16, r16)
        accm_a = _sc_scatter(msg_r, dst_r, zeros_acc)
        accz_a = _sc_scatter(zb_r, dst_r, zeros_acc)

        xp = _run_epi(accm_p, accz_p, xp, *epi_ws(l, "paper"))
        xa = _run_epi(accm_a, accz_a, xa, *epi_ws(l, "author"))

    return (xp, xa)


# speedup vs baseline: 1.0476x; 1.0476x over previous
import functools

import jax
import jax.numpy as jnp
import numpy as np
from jax import lax
from jax.experimental import pallas as pl
from jax.experimental.pallas import tpu as pltpu
from jax.experimental.pallas import tpu_sc as plsc

H = 8
D = 16
HID = 128
N = 10000
E = 320000
WID = HID + 16

NW = 32
CHUNK = 128
E_PAD = 327680
ROWS_PER_W = E_PAD // NW // CHUNK
NPAD = 10112
RPT = NPAD // 16
_CH = ((0, 128), (128, 128), (256, 128), (384, 128), (512, 120))

_f32 = jnp.float32
_i32 = jnp.int32


def _gelu(x):
    return 0.5 * x * (1.0 + lax.erf(x / np.sqrt(2.0).astype(np.float32)))


def _proj0_body(x0p, x0a, lwp, lbp, lwa, lba,
                wqp, bqp, wka, bka, wva, bva,
                wqa, bqa, wkp, bkp, wvp, bvp,
                xp, xa, qp, ka, va, qa, kp, vp):
    xpv = jnp.maximum(jnp.dot(x0p[...], lwp[...], preferred_element_type=_f32)
                      + lbp[...], 0.0)
    xav = jnp.maximum(jnp.dot(x0a[...], lwa[...], preferred_element_type=_f32)
                      + lba[...], 0.0)
    xp[...] = xpv
    xa[...] = xav
    qp[...] = jnp.dot(xpv, wqp[...], preferred_element_type=_f32) + bqp[...]
    ka[...] = jnp.dot(xav, wka[...], preferred_element_type=_f32) + bka[...]
    va[...] = jnp.dot(xav, wva[...], preferred_element_type=_f32) + bva[...]
    qa[...] = jnp.dot(xav, wqa[...], preferred_element_type=_f32) + bqa[...]
    kp[...] = jnp.dot(xpv, wkp[...], preferred_element_type=_f32) + bkp[...]
    vp[...] = jnp.dot(xpv, wvp[...], preferred_element_type=_f32) + bvp[...]


def _proj1_body(x0p, x0a,
                wqp, bqp, wka, bka, wva, bva,
                wqa, bqa, wkp, bkp, wvp, bvp,
                qp, ka, va, qa, kp, vp):
    xpv = x0p[...]
    xav = x0a[...]
    qp[...] = jnp.dot(xpv, wqp[...], preferred_element_type=_f32) + bqp[...]
    ka[...] = jnp.dot(xav, wka[...], preferred_element_type=_f32) + bka[...]
    va[...] = jnp.dot(xav, wva[...], preferred_element_type=_f32) + bva[...]
    qa[...] = jnp.dot(xav, wqa[...], preferred_element_type=_f32) + bqa[...]
    kp[...] = jnp.dot(xpv, wkp[...], preferred_element_type=_f32) + bkp[...]
    vp[...] = jnp.dot(xpv, wvp[...], preferred_element_type=_f32) + bvp[...]


_BM = 2000


def _run_proj0(x0p, x0a, ws):
    grid = (N // _BM,)
    xspec = pl.BlockSpec((_BM, HID), lambda i: (i, 0))
    wspec = pl.BlockSpec((HID, HID), lambda i: (0, 0))
    bspec = pl.BlockSpec((1, HID), lambda i: (0, 0))
    in_specs = [xspec, xspec] + [wspec if w.shape == (HID, HID) else bspec
                                 for w in ws]
    return pl.pallas_call(
        _proj0_body, grid=grid, in_specs=in_specs, out_specs=[xspec] * 8,
        out_shape=[jax.ShapeDtypeStruct((N, HID), _f32)] * 8)(x0p, x0a, *ws)


def _run_proj1(xp, xa, ws):
    grid = (N // _BM,)
    xspec = pl.BlockSpec((_BM, HID), lambda i: (i, 0))
    wspec = pl.BlockSpec((HID, HID), lambda i: (0, 0))
    bspec = pl.BlockSpec((1, HID), lambda i: (0, 0))
    in_specs = [xspec, xspec] + [wspec if w.shape == (HID, HID) else bspec
                                 for w in ws]
    return pl.pallas_call(
        _proj1_body, grid=grid, in_specs=in_specs, out_specs=[xspec] * 6,
        out_shape=[jax.ShapeDtypeStruct((N, HID), _f32)] * 6)(xp, xa, *ws)


_BE = 2048


def _edge_body(ke, qe, ve, e16, r16, msg, z128):
    p = ke[...] * qe[...]
    z = jnp.exp(jnp.dot(p, e16[...], preferred_element_type=_f32))
    zb = jnp.dot(z, r16[...], preferred_element_type=_f32)
    z128[...] = zb
    msg[...] = ve[...] * zb


def _run_edge(ke, qe, ve, e16, r16):
    grid = (E_PAD // _BE,)
    espec = pl.BlockSpec((_BE, HID), lambda i: (i, 0))
    return pl.pallas_call(
        _edge_body, grid=grid,
        in_specs=[espec, espec, espec,
                  pl.BlockSpec((HID, 16), lambda i: (0, 0)),
                  pl.BlockSpec((16, HID), lambda i: (0, 0))],
        out_specs=[espec, espec],
        out_shape=[jax.ShapeDtypeStruct((E_PAD, HID), _f32)] * 2)(ke, qe, ve,
                                                                  e16, r16)


def _epi_body(accm2, accz2, xin, aw, ab, skc, gbn, bbn, out):
    a = accm2[0:N, :] + accm2[NPAD:NPAD + N, :]
    szb = accz2[0:N, :] + accz2[NPAD:NPAD + N, :]
    aggn = a / (szb + 1e-16)
    g = _gelu(aggn)
    o = (jnp.dot(g, aw[...], preferred_element_type=_f32) + ab[...]
         + skc[...] * xin[...])
    h = _gelu(o) + xin[...]
    mu = jnp.mean(h, axis=0, keepdims=True)
    var = jnp.mean((h - mu) * (h - mu), axis=0, keepdims=True)
    out[...] = (h - mu) / jnp.sqrt(var + 1e-5) * gbn[...] + bbn[...]


def _run_epi(accm2, accz2, xin, aw, ab, skc, gbn, bbn):
    return pl.pallas_call(
        _epi_body,
        out_shape=jax.ShapeDtypeStruct((N, HID), _f32),
    )(accm2, accz2, xin, aw, ab, skc, gbn, bbn)


@functools.lru_cache(maxsize=None)
def _get_sc_gather():
    mesh = plsc.VectorSubcoreMesh(core_axis_name="c", subcore_axis_name="s")
    return functools.partial(
        pl.kernel,
        out_type=[jax.ShapeDtypeStruct((E_PAD, HID), _f32)] * 3,
        mesh=mesh,
        scratch_types=[
            pltpu.VMEM((1, CHUNK), _i32),
            pltpu.VMEM((1, CHUNK), _i32),
            pltpu.VMEM((CHUNK, HID), _f32),
            pltpu.VMEM((CHUNK, HID), _f32),
            pltpu.VMEM((CHUNK, HID), _f32),
            pltpu.SemaphoreType.DMA,
            pltpu.SemaphoreType.DMA,
            pltpu.SemaphoreType.DMA,
        ],
    )(_sc_gather_body)


def _sc_gather(ktab, vtab, qtab, src2d, dst2d):
    return _get_sc_gather()(ktab, vtab, qtab, src2d, dst2d)


def _sc_gather_body(ktab, vtab, qtab, src2d, dst2d, ke, ve, qe,
                    sidx, didx, kr, vr, qr, sem1, sem2, sem3):
    c = lax.axis_index("c")
    s = lax.axis_index("s")
    wid = c * 16 + s

    def body(i, carry):
        row = wid * ROWS_PER_W + i
        base = row * CHUNK
        pltpu.sync_copy(src2d.at[pl.ds(row, 1)], sidx)
        pltpu.sync_copy(dst2d.at[pl.ds(row, 1)], didx)
        ck = pltpu.async_copy(ktab.at[sidx.at[0]], kr, sem1)
        cv = pltpu.async_copy(vtab.at[sidx.at[0]], vr, sem2)
        cq = pltpu.async_copy(qtab.at[didx.at[0]], qr, sem3)
        ck.wait()
        cv.wait()
        cq.wait()
        pltpu.sync_copy(kr, ke.at[pl.ds(base, CHUNK)])
        pltpu.sync_copy(vr, ve.at[pl.ds(base, CHUNK)])
        pltpu.sync_copy(qr, qe.at[pl.ds(base, CHUNK)])
        return carry

    lax.fori_loop(0, ROWS_PER_W, body, 0)


@functools.lru_cache(maxsize=None)
def _get_sc_scatter():
    mesh = plsc.VectorSubcoreMesh(core_axis_name="c", subcore_axis_name="s")
    return functools.partial(
        pl.kernel,
        out_type=jax.ShapeDtypeStruct((2 * NPAD, HID), _f32),
        mesh=mesh,
        scratch_types=[
            pltpu.VMEM_SHARED((NPAD, HID), _f32),
            pltpu.VMEM((CHUNK, HID), _f32),
            pltpu.VMEM((CHUNK, HID), _f32),
            pltpu.VMEM((CHUNK,), _i32),
            pltpu.VMEM((CHUNK,), _i32),
            pltpu.SemaphoreType.DMA,
            pltpu.SemaphoreType.DMA,
        ],
    )(_sc_scatter_body)


def _sc_scatter(mz, dst2d, zeros_acc):
    return _get_sc_scatter()(mz, dst2d.reshape(-1), zeros_acc)


def _sc_scatter_body(mz, dst1d, zeros_acc, accout, acc, mv0, mv1, di0, di1,
                     semL0, semL1):
    c = lax.axis_index("c")
    s = lax.axis_index("s")
    wid = c * 16 + s
    r0 = s * RPT
    pltpu.sync_copy(zeros_acc.at[pl.ds(0, CHUNK)], mv0)
    for (o, n) in _CH:
        pltpu.sync_copy(mv0.at[pl.ds(0, n)], acc.at[pl.ds(r0 + o, n)])
    plsc.subcore_barrier()

    base0 = pl.multiple_of(wid * ROWS_PER_W * CHUNK, CHUNK)
    pltpu.async_copy(dst1d.at[pl.ds(base0, CHUNK)], di0, semL0)
    pltpu.async_copy(mz.at[pl.ds(base0, CHUNK)], mv0, semL0)

    bufs = ((mv0, di0, semL0), (mv1, di1, semL1))

    def body(j, carry):
        for b in (0, 1):
            i = j * 2 + b
            mv, di, semL = bufs[b]
            nmv, ndi, nsemL = bufs[1 - b]
            nbase = pl.multiple_of((wid * ROWS_PER_W + i + 1) * CHUNK, CHUNK)

            @pl.when(i + 1 < ROWS_PER_W)
            def _():
                pltpu.async_copy(dst1d.at[pl.ds(nbase, CHUNK)], ndi, nsemL)
                pltpu.async_copy(mz.at[pl.ds(nbase, CHUNK)], nmv, nsemL)

            pltpu.make_async_copy(dst1d.at[pl.ds(base0, CHUNK)], di,
                                  semL).wait()
            pltpu.make_async_copy(mz.at[pl.ds(base0, CHUNK)], mv, semL).wait()
            pltpu.sync_copy(mv, acc.at[di], add=True)
        return carry

    lax.fori_loop(0, ROWS_PER_W // 2, body, 0)
    plsc.subcore_barrier()
    out0 = c * NPAD + r0
    for (o, n) in _CH:
        pltpu.sync_copy(acc.at[pl.ds(r0 + o, n)], mv0.at[pl.ds(0, n)])
        pltpu.sync_copy(mv0.at[pl.ds(0, n)], accout.at[pl.ds(out0 + o, n)])


def _fold_rel(w, b, rel):
    w3 = w.reshape(HID, H, D)
    wf = jnp.einsum("chd,hde->che", w3, rel).reshape(HID, HID)
    bf = jnp.einsum("hd,hde->he", b.reshape(H, D), rel).reshape(HID)
    return wf, bf


def _pad_idx(idx, fill):
    idx = idx.astype(_i32)
    pad = jnp.full((E_PAD - E,), fill, _i32)
    return jnp.concatenate([idx, pad]).reshape(E_PAD // CHUNK, CHUNK)


def kernel(x_paper, x_author, edge_index_writes, edge_index_rev, params):
    p = params

    e16 = np.zeros((HID, 16), np.float32)
    r16 = np.zeros((16, HID), np.float32)
    for h in range(H):
        for d in range(D):
            e16[h * D + d, h] = 1.0
            r16[h, h * D + d] = 1.0
    e16 = jnp.asarray(e16)
    r16 = jnp.asarray(r16)

    zeros_acc = jnp.zeros((NPAD, HID), _f32)

    src_w = _pad_idx(edge_index_writes[0], 0)
    dst_w = _pad_idx(edge_index_writes[1], N)
    src_r = _pad_idx(edge_index_rev[0], 0)
    dst_r = _pad_idx(edge_index_rev[1], N)

    def b2(b):
        return b.reshape(1, HID)

    folded = {}
    for l in range(2):
        for (st, r, dt) in [("author", "writes", "paper"),
                            ("paper", "rev", "author")]:
            arel = p[f"arel_{r}_{l}"] * (p[f"prel_{r}_{l}"]
                                         / np.float32(np.sqrt(D)))[:, None, None]
            mrel = p[f"mrel_{r}_{l}"]
            folded[f"k_{r}_{l}"] = _fold_rel(p[f"k_w_{st}_{l}"],
                                             p[f"k_b_{st}_{l}"], arel)
            folded[f"v_{r}_{l}"] = _fold_rel(p[f"v_w_{st}_{l}"],
                                             p[f"v_b_{st}_{l}"], mrel)

    def proj_ws(l):
        kw_w, kb_w = folded[f"k_writes_{l}"]
        vw_w, vb_w = folded[f"v_writes_{l}"]
        kw_r, kb_r = folded[f"k_rev_{l}"]
        vw_r, vb_r = folded[f"v_rev_{l}"]
        return [p[f"q_w_paper_{l}"], b2(p[f"q_b_paper_{l}"]),
                kw_w, b2(kb_w), vw_w, b2(vb_w),
                p[f"q_w_author_{l}"], b2(p[f"q_b_author_{l}"]),
                kw_r, b2(kb_r), vw_r, b2(vb_r)]

    def epi_ws(l, t):
        sk = jax.nn.sigmoid(p[f"skip_{t}_{l}"])
        aw = sk * p[f"a_w_{t}_{l}"]
        ab = b2(sk * p[f"a_b_{t}_{l}"])
        skc = (1.0 - sk).reshape(1, 1)
        return aw, ab, skc, b2(p[f"bn_g_{t}_{l}"]), b2(p[f"bn_b_{t}_{l}"])

    xp, xa, qp, ka, va, qa, kp, vp = _run_proj0(
        x_paper, x_author,
        [p["lin_w_paper"], b2(p["lin_b_paper"]),
         p["lin_w_author"], b2(p["lin_b_author"])] + proj_ws(0))

    for l in range(2):
        if l == 1:
            qp, ka, va, qa, kp, vp = _run_proj1(xp, xa, proj_ws(1))

        ke_w, ve_w, qe_w = _sc_gather(ka, va, qp, src_w, dst_w)
        msg_w, zb_w = _run_edge(ke_w, qe_w, ve_w, e16, r16)
        accm_p = _sc_scatter(msg_w, dst_w, zeros_acc)
        accz_p = _sc_scatter(zb_w, dst_w, zeros_acc)

        ke_r, ve_r, qe_r = _sc_gather(kp, vp, qa, src_r, dst_r)
        msg_r, zb_r = _run_edge(ke_r, qe_r, ve_r, e16, r16)
        accm_a = _sc_scatter(msg_r, dst_r, zeros_acc)
        accz_a = _sc_scatter(zb_r, dst_r, zeros_acc)

        xp = _run_epi(accm_p, accz_p, xp, *epi_ws(l, "paper"))
        xa = _run_epi(accm_a, accz_a, xa, *epi_ws(l, "author"))

    return (xp, xa)
